# Initial kernel scaffold; baseline (speedup 1.0000x reference)
import jax
import jax.numpy as jnp
from jax.experimental import pallas as pl


def kernel(features, edge_index, W1, b1, W2, b2, W3, b3):
    raise NotImplementedError("write your pallas kernel here")


# speedup vs baseline: 2.4897x; 2.4897x over previous
import functools

import jax
import jax.numpy as jnp
from jax import lax
from jax.experimental import pallas as pl
from jax.experimental.pallas import tpu as pltpu
from jax.experimental.pallas import tpu_sc as plsc

N = 10000
D = 256
H = 256
C = 40
E = 160000

NC = 2
NS = 16
NPAD = 12800
EPAD = 163840
CPAD = 128

RB = 400
GN = N // RB
GP = NPAD // RB
SUBR = NPAD // NS

_mesh = plsc.VectorSubcoreMesh(core_axis_name="c", subcore_axis_name="s")


@functools.partial(
    pl.kernel,
    mesh=_mesh,
    out_type=jax.ShapeDtypeStruct((2 * NPAD,), jnp.float32),
    scratch_types=[
        pltpu.VMEM((80,), jnp.int32),
        pltpu.VMEM((80,), jnp.float32),
        pltpu.VMEM((SUBR,), jnp.float32),
        pltpu.VMEM_SHARED((NPAD,), jnp.float32),
    ],
)
def _deg_kernel(ei_hbm, out_hbm, idx_v, ones_v, buf_v, acc):
    c = lax.axis_index("c")
    s = lax.axis_index("s")

    @pl.loop(0, SUBR // 16)
    def _(i):
        buf_v[pl.ds(i * 16, 16)] = jnp.zeros((16,), jnp.float32)

    @pl.loop(0, 5)
    def _(i):
        ones_v[pl.ds(i * 16, 16)] = jnp.full((16,), 1.0, jnp.float32)

    pltpu.sync_copy(buf_v, acc.at[pl.ds(s * SUBR, SUBR)])
    plsc.subcore_barrier()
    base = c * E + s * (E // NS)

    @pl.loop(0, (E // NS) // 80)
    def _(b):
        pltpu.sync_copy(ei_hbm.at[pl.ds(base + b * 80, 80)], idx_v)
        pltpu.sync_copy(ones_v, acc.at[idx_v], add=True)

    plsc.subcore_barrier()
    pltpu.sync_copy(acc.at[pl.ds(s * SUBR, SUBR)], buf_v)
    pltpu.sync_copy(buf_v, out_hbm.at[pl.ds(c * NPAD + s * SUBR, SUBR)])


def _make_agg(width, split_features):
    nb = (EPAD // NS if split_features else EPAD // (NC * NS)) // 128

    @functools.partial(
        pl.kernel,
        mesh=_mesh,
        out_type=jax.ShapeDtypeStruct((2 * NPAD, width), jnp.float32),
        scratch_types=[
            pltpu.VMEM((128,), jnp.int32),
            pltpu.VMEM((128,), jnp.int32),
            pltpu.VMEM((128, width), jnp.float32),
            pltpu.VMEM_SHARED((NPAD, width), jnp.float32),
        ],
    )
    def agg(table_hbm, src_hbm, dst_hbm, zeros_hbm, out_hbm, sidx, didx, rows, acc):
        c = lax.axis_index("c")
        s = lax.axis_index("s")
        pltpu.sync_copy(zeros_hbm.at[pl.ds(s * SUBR, SUBR)],
                        acc.at[pl.ds(s * SUBR, SUBR)])
        plsc.subcore_barrier()
        if split_features:
            dst_base = s * (EPAD // NS)
            src_base = c * EPAD + dst_base
        else:
            dst_base = (c * NS + s) * (EPAD // (NC * NS))
            src_base = dst_base

        @pl.loop(0, nb)
        def _(b):
            pltpu.sync_copy(src_hbm.at[pl.ds(src_base + b * 128, 128)], sidx)
            pltpu.sync_copy(dst_hbm.at[pl.ds(dst_base + b * 128, 128)], didx)
            pltpu.sync_copy(table_hbm.at[sidx], rows)
            pltpu.sync_copy(rows, acc.at[didx], add=True)

        plsc.subcore_barrier()
        pltpu.sync_copy(acc.at[pl.ds(s * SUBR, SUBR)],
                        out_hbm.at[pl.ds(c * NPAD + s * SUBR, SUBR)])

    return agg


_agg128 = _make_agg(128, True)
_agg64 = _make_agg(CPAD, False)


def _norms_from(deg_blk):
    d = deg_blk
    return jnp.where(d > 0, lax.rsqrt(jnp.maximum(d, 1e-12)), 0.0)


def _mm_body(x_ref, w_ref, o_ref):
    o_ref[...] = jnp.dot(x_ref[...], w_ref[...],
                         preferred_element_type=jnp.float32)


def _tc_matmul(x, w):
    return pl.pallas_call(
        _mm_body,
        grid=(GN,),
        in_specs=[pl.BlockSpec((RB, D), lambda i: (i, 0)),
                  pl.BlockSpec((D, H), lambda i: (0, 0))],
        out_specs=pl.BlockSpec((RB, H), lambda i: (i, 0)),
        out_shape=jax.ShapeDtypeStruct((N, H), jnp.float32),
    )(x, w)


def _scale_split_body(m_ref, degs_ref, o_ref):
    o_ref[...] = m_ref[...] * _norms_from(degs_ref[...])


def _tc_scale_split(m1, deg):
    return pl.pallas_call(
        _scale_split_body,
        grid=(GN, 2),
        in_specs=[pl.BlockSpec((RB, 128), lambda i, j: (i, j)),
                  pl.BlockSpec((RB, 1), lambda i, j: (i, 0))],
        out_specs=pl.BlockSpec((RB, 128), lambda i, j: (j * GN + i, 0)),
        out_shape=jax.ShapeDtypeStruct((2 * N, 128), jnp.float32),
    )(m1, deg)


def _layer_body(aggA_ref, aggB_ref, degs_ref, degd_ref, b_ref, w_ref, o_ref):
    nd = _norms_from(degd_ref[...])
    ns = _norms_from(degs_ref[...])
    x = jnp.concatenate([aggA_ref[...], aggB_ref[...]], axis=1)
    x = jax.nn.relu(x * nd + b_ref[...])
    o_ref[...] = jnp.dot(x, w_ref[...], preferred_element_type=jnp.float32) * ns


def _tc_layer(agg, deg, b_row, w):
    return pl.pallas_call(
        _layer_body,
        grid=(GN, 2),
        in_specs=[pl.BlockSpec((RB, 128), lambda i, j: (i, 0)),
                  pl.BlockSpec((RB, 128), lambda i, j: (GP + i, 0)),
                  pl.BlockSpec((RB, 1), lambda i, j: (i, 0)),
                  pl.BlockSpec((RB, 1), lambda i, j: (GP + i, 0)),
                  pl.BlockSpec((1, H), lambda i, j: (0, 0)),
                  pl.BlockSpec((H, 128), lambda i, j: (0, j))],
        out_specs=pl.BlockSpec((RB, 128), lambda i, j: (j * GN + i, 0)),
        out_shape=jax.ShapeDtypeStruct((2 * N, 128), jnp.float32),
    )(agg, agg, deg, deg, b_row, w)


def _layer3_body(aggA_ref, aggB_ref, degs_ref, degd_ref, b_ref, w_ref, o_ref):
    nd = _norms_from(degd_ref[...])
    ns = _norms_from(degs_ref[...])
    x = jnp.concatenate([aggA_ref[...], aggB_ref[...]], axis=1)
    x = jax.nn.relu(x * nd + b_ref[...])
    o_ref[...] = jnp.dot(x, w_ref[...], preferred_element_type=jnp.float32) * ns


def _tc_layer3(agg, deg, b_row, w_pad):
    return pl.pallas_call(
        _layer3_body,
        grid=(GN,),
        in_specs=[pl.BlockSpec((RB, 128), lambda i: (i, 0)),
                  pl.BlockSpec((RB, 128), lambda i: (GP + i, 0)),
                  pl.BlockSpec((RB, 1), lambda i: (i, 0)),
                  pl.BlockSpec((RB, 1), lambda i: (GP + i, 0)),
                  pl.BlockSpec((1, H), lambda i: (0, 0)),
                  pl.BlockSpec((H, CPAD), lambda i: (0, 0))],
        out_specs=pl.BlockSpec((RB, CPAD), lambda i: (i, 0)),
        out_shape=jax.ShapeDtypeStruct((N, CPAD), jnp.float32),
    )(agg, agg, deg, deg, b_row, w_pad)


def _final_body(p0_ref, p1_ref, degd_ref, b_ref, o_ref):
    nd = _norms_from(degd_ref[...])
    full = (p0_ref[...] + p1_ref[...]) * nd + b_ref[...]
    o_ref[...] = full[:, :C]


def _tc_final(p3, deg, b3_row):
    return pl.pallas_call(
        _final_body,
        grid=(GN,),
        in_specs=[pl.BlockSpec((RB, CPAD), lambda i: (i, 0)),
                  pl.BlockSpec((RB, CPAD), lambda i: (GP + i, 0)),
                  pl.BlockSpec((RB, 1), lambda i: (GP + i, 0)),
                  pl.BlockSpec((1, CPAD), lambda i: (0, 0))],
        out_specs=pl.BlockSpec((RB, C), lambda i: (i, 0)),
        out_shape=jax.ShapeDtypeStruct((N, C), jnp.float32),
    )(p3, p3, deg, b3_row)


def kernel(features, edge_index, W1, b1, W2, b2, W3, b3):
    src = edge_index[0]
    dst = edge_index[1]
    pad = EPAD - E
    src_p = jnp.concatenate([src, jnp.zeros((pad,), jnp.int32)])
    dst_p = jnp.concatenate([dst, jnp.full((pad,), N, jnp.int32)])
    src2 = jnp.concatenate([src_p, src_p + N])
    ei_flat = jnp.concatenate([src, dst])

    z128 = jnp.zeros((NPAD, 128), jnp.float32)
    z64 = z128

    b1_row = b1.reshape(1, H)
    b2_row = b2.reshape(1, H)
    b3_row = jnp.pad(b3, (0, CPAD - C)).reshape(1, CPAD)
    W3p = jnp.pad(W3, ((0, 0), (0, CPAD - C)))

    deg = _deg_kernel(ei_flat).reshape(2 * NPAD, 1)
    m1 = _tc_matmul(features, W1)
    h1 = _tc_scale_split(m1, deg)
    a1 = _agg128(h1, src2, dst_p, z128)
    h2 = _tc_layer(a1, deg, b1_row, W2)
    a2 = _agg128(h2, src2, dst_p, z128)
    h3 = _tc_layer3(a2, deg, b2_row, W3p)
    p3 = _agg64(h3, src_p, dst_p, z64)
    return _tc_final(p3, deg, b3_row)
